# baseline (speedup 1.0000x reference)
import jax
import jax.numpy as jnp
from jax.experimental import pallas as pl
from jax.experimental.pallas import tpu as pltpu

_CB = 16


def _maxcopy_kernel(x_ref, max_ref, out_ref):
    blk = x_ref[...]
    max_ref[0, 0, :] = jnp.max(blk, axis=(0, 2, 3))
    out_ref[...] = blk


def _unsel_kernel(v_ref, u_ref, *, k):
    v = v_ref[...]
    b_, c_ = v.shape
    vj = v[:, None, :]
    vc = v[:, :, None]
    j = jax.lax.broadcasted_iota(jnp.int32, (1, c_, c_), 2)
    c = jax.lax.broadcasted_iota(jnp.int32, (1, c_, c_), 1)
    beats = (vj > vc) | ((vj == vc) & (j < c))
    rank = jnp.sum(beats.astype(jnp.int32), axis=2)
    slot = jax.lax.broadcasted_iota(jnp.int32, (b_, c_, c_ - k), 2)
    chan = jax.lax.broadcasted_iota(jnp.int32, (b_, c_, c_ - k), 1)
    onehot = (rank[:, :, None] - k) == slot
    u_ref[...] = jnp.sum(jnp.where(onehot, chan, 0), axis=1)


def _zero_kernel(u_ref, buf_ref, out_ref):
    del u_ref, buf_ref
    out_ref[...] = jnp.zeros_like(out_ref)


def kernel(x):
    B, C, H, W = x.shape
    k = C // 2
    ncb = C // _CB

    maxes, out1 = pl.pallas_call(
        _maxcopy_kernel,
        grid=(B, ncb),
        in_specs=[pl.BlockSpec((1, _CB, H, W), lambda b, i: (b, i, 0, 0))],
        out_specs=[
            pl.BlockSpec((1, 1, _CB), lambda b, i: (b * ncb + i, 0, 0)),
            pl.BlockSpec((1, _CB, H, W), lambda b, i: (b, i, 0, 0)),
        ],
        out_shape=[
            jax.ShapeDtypeStruct((B * ncb, 1, _CB), x.dtype),
            jax.ShapeDtypeStruct((B, C, H, W), x.dtype),
        ],
    )(x)

    v = maxes.reshape(B, C)

    unsel = pl.pallas_call(
        lambda v_ref, u_ref: _unsel_kernel(v_ref, u_ref, k=k),
        in_specs=[pl.BlockSpec((B, C), lambda: (0, 0))],
        out_specs=pl.BlockSpec((B, C - k), lambda: (0, 0)),
        out_shape=jax.ShapeDtypeStruct((B, C - k), jnp.int32),
    )(v)

    grid_spec = pltpu.PrefetchScalarGridSpec(
        num_scalar_prefetch=1,
        grid=(B, C - k),
        in_specs=[pl.BlockSpec(memory_space=pl.ANY)],
        out_specs=pl.BlockSpec((1, 1, H, W), lambda b, j, u: (b, u[b, j], 0, 0)),
    )
    out = pl.pallas_call(
        _zero_kernel,
        grid_spec=grid_spec,
        out_shape=jax.ShapeDtypeStruct((B, C, H, W), x.dtype),
        input_output_aliases={1: 0},
    )(unsel, out1)

    return out


# speedup vs baseline: 1.7794x; 1.7794x over previous
import jax
import jax.numpy as jnp
from jax.experimental import pallas as pl
from jax.experimental.pallas import tpu as pltpu

_CB = 16
_NSLOT = 16


def _maxcopy_kernel(x_ref, max_ref, out_ref):
    blk = x_ref[...]
    max_ref[0, 0, :] = jnp.max(blk, axis=(0, 2, 3))
    out_ref[...] = blk


def _unsel_kernel(v_ref, u_ref, *, k):
    v = v_ref[...]
    b_, c_ = v.shape
    vj = v[:, None, :]
    vc = v[:, :, None]
    j = jax.lax.broadcasted_iota(jnp.int32, (1, c_, c_), 2)
    c = jax.lax.broadcasted_iota(jnp.int32, (1, c_, c_), 1)
    beats = (vj > vc) | ((vj == vc) & (j < c))
    rank = jnp.sum(beats.astype(jnp.int32), axis=2)
    slot = jax.lax.broadcasted_iota(jnp.int32, (b_, c_, c_ - k), 2)
    chan = jax.lax.broadcasted_iota(jnp.int32, (b_, c_, c_ - k), 1)
    onehot = (rank[:, :, None] - k) == slot
    u_ref[...] = jnp.sum(jnp.where(onehot, chan, 0), axis=1)


def _make_zero_kernel(B, C, H, W, k):
    nunsel = C - k
    total = B * nunsel
    ngroups = total // _NSLOT

    def _zero_kernel(u_ref, buf_ref, out_ref, zeros_ref, sem_ref):
        del buf_ref
        zeros_ref[...] = jnp.zeros((H, W), dtype=zeros_ref.dtype)

        def _issue(g, bank):
            for s in range(_NSLOT):
                i = g * _NSLOT + s
                b = i // nunsel
                j = jax.lax.rem(i, nunsel)
                ch = u_ref[b, j]
                pltpu.make_async_copy(
                    zeros_ref, out_ref.at[b, ch], sem_ref.at[bank * _NSLOT + s]
                ).start()

        def _wait(bank):
            for s in range(_NSLOT):
                pltpu.make_async_copy(
                    zeros_ref, out_ref.at[0, 0], sem_ref.at[bank * _NSLOT + s]
                ).wait()

        def body(g, carry):
            bank = jax.lax.rem(g, 2)
            _issue(g, bank)

            @pl.when(g > 0)
            def _():
                _wait(1 - bank)

            return carry

        jax.lax.fori_loop(0, ngroups, body, 0)
        _wait((ngroups - 1) % 2)

    return _zero_kernel


def kernel(x):
    B, C, H, W = x.shape
    k = C // 2
    ncb = C // _CB

    maxes, out1 = pl.pallas_call(
        _maxcopy_kernel,
        grid=(B, ncb),
        in_specs=[pl.BlockSpec((1, _CB, H, W), lambda b, i: (b, i, 0, 0))],
        out_specs=[
            pl.BlockSpec((1, 1, _CB), lambda b, i: (b * ncb + i, 0, 0)),
            pl.BlockSpec((1, _CB, H, W), lambda b, i: (b, i, 0, 0)),
        ],
        out_shape=[
            jax.ShapeDtypeStruct((B * ncb, 1, _CB), x.dtype),
            jax.ShapeDtypeStruct((B, C, H, W), x.dtype),
        ],
    )(x)

    v = maxes.reshape(B, C)

    unsel = pl.pallas_call(
        lambda v_ref, u_ref: _unsel_kernel(v_ref, u_ref, k=k),
        in_specs=[pl.BlockSpec((B, C), lambda: (0, 0))],
        out_specs=pl.BlockSpec((B, C - k), lambda: (0, 0)),
        out_shape=jax.ShapeDtypeStruct((B, C - k), jnp.int32),
    )(v)

    out = pl.pallas_call(
        _make_zero_kernel(B, C, H, W, k),
        in_specs=[
            pl.BlockSpec(memory_space=pltpu.SMEM),
            pl.BlockSpec(memory_space=pl.ANY),
        ],
        out_specs=pl.BlockSpec(memory_space=pl.ANY),
        out_shape=jax.ShapeDtypeStruct((B, C, H, W), x.dtype),
        scratch_shapes=[
            pltpu.VMEM((H, W), x.dtype),
            pltpu.SemaphoreType.DMA((2 * _NSLOT,)),
        ],
        input_output_aliases={1: 0},
    )(unsel, out1)

    return out


# speedup vs baseline: 1.8046x; 1.0141x over previous
import jax
import jax.numpy as jnp
from jax.experimental import pallas as pl
from jax.experimental.pallas import tpu as pltpu

_CB = 16


def _make_fused_kernel(B, C, H, W, k):
    nc = C // _CB

    def _fused(x_ref, out_ref, acc_ref, maxv_ref, rank_ref, rks_ref,
               insem, outsem, rksem):
        b = pl.program_id(0)

        def in_cp(i):
            return pltpu.make_async_copy(
                x_ref.at[b, pl.ds(i * _CB, _CB)],
                acc_ref.at[pl.ds(i * _CB, _CB)],
                insem.at[i],
            )

        def out_cp(i):
            return pltpu.make_async_copy(
                acc_ref.at[pl.ds(i * _CB, _CB)],
                out_ref.at[b, pl.ds(i * _CB, _CB)],
                outsem.at[i],
            )

        for i in range(nc):
            @pl.when(b > 0)
            def _(i=i):
                out_cp(i).wait()

            in_cp(i).start()

        for i in range(nc):
            in_cp(i).wait()
            blk = acc_ref[i * _CB:(i + 1) * _CB]
            maxv_ref[0, i * _CB:(i + 1) * _CB] = jnp.max(blk, axis=(1, 2))

        v = maxv_ref[...]
        vj = v[:, None, :]
        vc = v[:, :, None]
        j = jax.lax.broadcasted_iota(jnp.int32, (1, C, C), 2)
        c = jax.lax.broadcasted_iota(jnp.int32, (1, C, C), 1)
        beats = (vj > vc) | ((vj == vc) & (j < c))
        rank_ref[...] = jnp.sum(beats.astype(jnp.int32), axis=2)

        cp = pltpu.make_async_copy(rank_ref, rks_ref, rksem)
        cp.start()
        cp.wait()

        def zero_body(ch, carry):
            @pl.when(rks_ref[0, ch] >= k)
            def _():
                acc_ref[pl.ds(ch, 1)] = jnp.zeros((1, H, W), acc_ref.dtype)
            return carry

        jax.lax.fori_loop(0, C, zero_body, 0)

        for i in range(nc):
            out_cp(i).start()

        @pl.when(b == B - 1)
        def _():
            for i in range(nc):
                out_cp(i).wait()

    return _fused


def kernel(x):
    B, C, H, W = x.shape
    k = C // 2

    out = pl.pallas_call(
        _make_fused_kernel(B, C, H, W, k),
        grid=(B,),
        in_specs=[pl.BlockSpec(memory_space=pl.ANY)],
        out_specs=pl.BlockSpec(memory_space=pl.ANY),
        out_shape=jax.ShapeDtypeStruct((B, C, H, W), x.dtype),
        scratch_shapes=[
            pltpu.VMEM((C, H, W), x.dtype),
            pltpu.VMEM((1, C), x.dtype),
            pltpu.VMEM((1, C), jnp.int32),
            pltpu.SMEM((1, C), jnp.int32),
            pltpu.SemaphoreType.DMA((C // _CB,)),
            pltpu.SemaphoreType.DMA((C // _CB,)),
            pltpu.SemaphoreType.DMA,
        ],
    )(x)

    return out


# speedup vs baseline: 2.1354x; 1.1833x over previous
import jax
import jax.numpy as jnp
from jax.experimental import pallas as pl
from jax.experimental.pallas import tpu as pltpu

_CB = 16


def _make_fused_kernel(B, C, H, W, k):
    nc = C // _CB

    def _fused(x_ref, out_ref, acc_ref, zeros_ref, maxv_ref, rank_ref,
               rks_ref, insem, outsem, rksem):
        b = pl.program_id(0)

        def in_cp(i):
            return pltpu.make_async_copy(
                x_ref.at[b, pl.ds(i * _CB, _CB)],
                acc_ref.at[pl.ds(i * _CB, _CB)],
                insem.at[i],
            )

        def out_chunk_cp(i):
            return pltpu.make_async_copy(
                acc_ref.at[pl.ds(i * _CB, _CB)],
                out_ref.at[b, pl.ds(i * _CB, _CB)],
                outsem.at[i],
            )

        @pl.when(b == 0)
        def _():
            zeros_ref[...] = jnp.zeros((1, H, W), zeros_ref.dtype)

        for i in range(nc):
            @pl.when(b > 0)
            def _(i=i):
                out_chunk_cp(i).wait()

            in_cp(i).start()

        for i in range(nc):
            in_cp(i).wait()
            blk = acc_ref[i * _CB:(i + 1) * _CB]
            maxv_ref[0, i * _CB:(i + 1) * _CB] = jnp.max(blk, axis=(1, 2))

        v = maxv_ref[...]
        vj = v[:, None, :]
        vc = v[:, :, None]
        j = jax.lax.broadcasted_iota(jnp.int32, (1, C, C), 2)
        c = jax.lax.broadcasted_iota(jnp.int32, (1, C, C), 1)
        beats = (vj > vc) | ((vj == vc) & (j < c))
        rank_ref[...] = jnp.sum(beats.astype(jnp.int32), axis=2)

        cp = pltpu.make_async_copy(rank_ref, rks_ref, rksem)
        cp.start()
        cp.wait()

        def wb_body(ch, carry):
            sel = rks_ref[0, ch] < k
            sem = outsem.at[ch // _CB]
            dst = out_ref.at[b, pl.ds(ch, 1)]

            @pl.when(sel)
            def _():
                pltpu.make_async_copy(acc_ref.at[pl.ds(ch, 1)], dst, sem).start()

            @pl.when(jnp.logical_not(sel))
            def _():
                pltpu.make_async_copy(zeros_ref, dst, sem).start()

            return carry

        jax.lax.fori_loop(0, C, wb_body, 0)

        @pl.when(b == B - 1)
        def _():
            for i in range(nc):
                out_chunk_cp(i).wait()

    return _fused


def kernel(x):
    B, C, H, W = x.shape
    k = C // 2

    out = pl.pallas_call(
        _make_fused_kernel(B, C, H, W, k),
        grid=(B,),
        in_specs=[pl.BlockSpec(memory_space=pl.ANY)],
        out_specs=pl.BlockSpec(memory_space=pl.ANY),
        out_shape=jax.ShapeDtypeStruct((B, C, H, W), x.dtype),
        scratch_shapes=[
            pltpu.VMEM((C, H, W), x.dtype),
            pltpu.VMEM((1, H, W), x.dtype),
            pltpu.VMEM((1, C), x.dtype),
            pltpu.VMEM((1, C), jnp.int32),
            pltpu.SMEM((1, C), jnp.int32),
            pltpu.SemaphoreType.DMA((C // _CB,)),
            pltpu.SemaphoreType.DMA((C // _CB,)),
            pltpu.SemaphoreType.DMA,
        ],
    )(x)

    return out


# speedup vs baseline: 2.1400x; 1.0022x over previous
import jax
import jax.numpy as jnp
from jax.experimental import pallas as pl
from jax.experimental.pallas import tpu as pltpu

_CB = 32


def _make_fused_kernel(B, C, H, W, k):
    nc = C // _CB

    def _fused(x_ref, out_ref, acc_ref, zeros_ref, maxv_ref, rank_ref,
               rks_ref, insem, outsem, rksem):
        b = pl.program_id(0)

        def in_cp(i):
            return pltpu.make_async_copy(
                x_ref.at[b, pl.ds(i * _CB, _CB)],
                acc_ref.at[pl.ds(i * _CB, _CB)],
                insem.at[i],
            )

        def out_chunk_cp(i):
            return pltpu.make_async_copy(
                acc_ref.at[pl.ds(i * _CB, _CB)],
                out_ref.at[b, pl.ds(i * _CB, _CB)],
                outsem.at[i],
            )

        @pl.when(b == 0)
        def _():
            zeros_ref[...] = jnp.zeros((1, H, W), zeros_ref.dtype)

        for i in range(nc):
            @pl.when(b > 0)
            def _(i=i):
                out_chunk_cp(i).wait()

            in_cp(i).start()

        for i in range(nc):
            in_cp(i).wait()
            blk = acc_ref[i * _CB:(i + 1) * _CB]
            maxv_ref[0, i * _CB:(i + 1) * _CB] = jnp.max(blk, axis=(1, 2))

        v = maxv_ref[...]
        vj = v[:, None, :]
        vc = v[:, :, None]
        j = jax.lax.broadcasted_iota(jnp.int32, (1, C, C), 2)
        c = jax.lax.broadcasted_iota(jnp.int32, (1, C, C), 1)
        beats = (vj > vc) | ((vj == vc) & (j < c))
        rank_ref[...] = jnp.sum(beats.astype(jnp.int32), axis=2)

        cp = pltpu.make_async_copy(rank_ref, rks_ref, rksem)
        cp.start()
        cp.wait()

        def wb_body(ch, carry):
            sel = rks_ref[0, ch] < k
            sem = outsem.at[ch // _CB]
            dst = out_ref.at[b, pl.ds(ch, 1)]

            @pl.when(sel)
            def _():
                pltpu.make_async_copy(acc_ref.at[pl.ds(ch, 1)], dst, sem).start()

            @pl.when(jnp.logical_not(sel))
            def _():
                pltpu.make_async_copy(zeros_ref, dst, sem).start()

            return carry

        jax.lax.fori_loop(0, C, wb_body, 0)

        @pl.when(b == B - 1)
        def _():
            for i in range(nc):
                out_chunk_cp(i).wait()

    return _fused


def kernel(x):
    B, C, H, W = x.shape
    k = C // 2

    out = pl.pallas_call(
        _make_fused_kernel(B, C, H, W, k),
        grid=(B,),
        in_specs=[pl.BlockSpec(memory_space=pl.ANY)],
        out_specs=pl.BlockSpec(memory_space=pl.ANY),
        out_shape=jax.ShapeDtypeStruct((B, C, H, W), x.dtype),
        scratch_shapes=[
            pltpu.VMEM((C, H, W), x.dtype),
            pltpu.VMEM((1, H, W), x.dtype),
            pltpu.VMEM((1, C), x.dtype),
            pltpu.VMEM((1, C), jnp.int32),
            pltpu.SMEM((1, C), jnp.int32),
            pltpu.SemaphoreType.DMA((C // _CB,)),
            pltpu.SemaphoreType.DMA((C // _CB,)),
            pltpu.SemaphoreType.DMA,
        ],
    )(x)

    return out
